# baseline (speedup 1.0000x reference)
import functools

import jax
import jax.numpy as jnp
from jax.experimental import pallas as pl
from jax.experimental.pallas import tpu as pltpu

_BLK = 1000


def _fused_gru_kernel(x_ref, wz_ref, bz_ref, wh_ref, bh_ref, wl_ref, bl_ref,
                      out_ref):
    x = x_ref[...]
    z = jax.nn.sigmoid(
        jnp.dot(x, wz_ref[...], preferred_element_type=jnp.float32)
        + bz_ref[...])
    ht = jnp.tanh(
        jnp.dot(x, wh_ref[...], preferred_element_type=jnp.float32)
        + bh_ref[...])
    h = jax.nn.relu((1.0 - z) * ht)
    out_ref[...] = (
        jnp.dot(h, wl_ref[...], preferred_element_type=jnp.float32)
        + bl_ref[...])


@functools.partial(jax.jit, static_argnames=())
def kernel(x, edge_index, edge_weight, W_xz, b_xz, W_hz, b_hz, W_xr, b_xr,
           W_hr, b_hr, W_xh, b_xh, W_hh, b_hh, W_lin, b_lin):
    n, f_in = x.shape
    out_len = W_lin.shape[1]
    bz = (b_xz + b_hz).reshape(1, -1)
    bh = (b_xh + b_hh).reshape(1, -1)
    bl = b_lin.reshape(1, -1)

    grid = (n // _BLK,)
    out = pl.pallas_call(
        _fused_gru_kernel,
        grid=grid,
        in_specs=[
            pl.BlockSpec((_BLK, f_in), lambda i: (i, 0)),
            pl.BlockSpec((f_in, W_xz.shape[1]), lambda i: (0, 0)),
            pl.BlockSpec((1, W_xz.shape[1]), lambda i: (0, 0)),
            pl.BlockSpec((f_in, W_xh.shape[1]), lambda i: (0, 0)),
            pl.BlockSpec((1, W_xh.shape[1]), lambda i: (0, 0)),
            pl.BlockSpec((W_lin.shape[0], out_len), lambda i: (0, 0)),
            pl.BlockSpec((1, out_len), lambda i: (0, 0)),
        ],
        out_specs=pl.BlockSpec((_BLK, out_len), lambda i: (i, 0)),
        out_shape=jax.ShapeDtypeStruct((n, out_len), x.dtype),
        compiler_params=pltpu.CompilerParams(
            dimension_semantics=("parallel",)),
    )(x, W_xz, bz, W_xh, bh, W_lin, bl)
    return (out,)


# speedup vs baseline: 1.1996x; 1.1996x over previous
import functools

import jax
import jax.numpy as jnp
from jax.experimental import pallas as pl
from jax.experimental.pallas import tpu as pltpu

_BLK = 2000


def _fused_gru_kernel(x_ref, wz_ref, bz_ref, wh_ref, bh_ref, wl_ref, bl_ref,
                      out_ref):
    x = x_ref[...]
    z = jax.nn.sigmoid(
        jnp.dot(x, wz_ref[...], preferred_element_type=jnp.float32)
        + bz_ref[...])
    ht = jnp.tanh(
        jnp.dot(x, wh_ref[...], preferred_element_type=jnp.float32)
        + bh_ref[...])
    h = jax.nn.relu((1.0 - z) * ht)
    out_ref[...] = (
        jnp.dot(h, wl_ref[...], preferred_element_type=jnp.float32)
        + bl_ref[...])


@functools.partial(jax.jit, static_argnames=())
def kernel(x, edge_index, edge_weight, W_xz, b_xz, W_hz, b_hz, W_xr, b_xr,
           W_hr, b_hr, W_xh, b_xh, W_hh, b_hh, W_lin, b_lin):
    n, f_in = x.shape
    out_len = W_lin.shape[1]
    bz = (b_xz + b_hz).reshape(1, -1)
    bh = (b_xh + b_hh).reshape(1, -1)
    bl = b_lin.reshape(1, -1)

    grid = (n // _BLK,)
    out = pl.pallas_call(
        _fused_gru_kernel,
        grid=grid,
        in_specs=[
            pl.BlockSpec((_BLK, f_in), lambda i: (i, 0)),
            pl.BlockSpec((f_in, W_xz.shape[1]), lambda i: (0, 0)),
            pl.BlockSpec((1, W_xz.shape[1]), lambda i: (0, 0)),
            pl.BlockSpec((f_in, W_xh.shape[1]), lambda i: (0, 0)),
            pl.BlockSpec((1, W_xh.shape[1]), lambda i: (0, 0)),
            pl.BlockSpec((W_lin.shape[0], out_len), lambda i: (0, 0)),
            pl.BlockSpec((1, out_len), lambda i: (0, 0)),
        ],
        out_specs=pl.BlockSpec((_BLK, out_len), lambda i: (i, 0)),
        out_shape=jax.ShapeDtypeStruct((n, out_len), x.dtype),
        compiler_params=pltpu.CompilerParams(
            dimension_semantics=("parallel",)),
    )(x, W_xz, bz, W_xh, bh, W_lin, bl)
    return (out,)


# speedup vs baseline: 1.2014x; 1.0015x over previous
import functools

import jax
import jax.numpy as jnp
from jax.experimental import pallas as pl
from jax.experimental.pallas import tpu as pltpu

_BLK = 5000


def _fused_gru_kernel(x_ref, wz_ref, bz_ref, wh_ref, bh_ref, wl_ref, bl_ref,
                      out_ref):
    x = x_ref[...]
    z = jax.nn.sigmoid(
        jnp.dot(x, wz_ref[...], preferred_element_type=jnp.float32)
        + bz_ref[...])
    ht = jnp.tanh(
        jnp.dot(x, wh_ref[...], preferred_element_type=jnp.float32)
        + bh_ref[...])
    h = jax.nn.relu((1.0 - z) * ht)
    out_ref[...] = (
        jnp.dot(h, wl_ref[...], preferred_element_type=jnp.float32)
        + bl_ref[...])


@functools.partial(jax.jit, static_argnames=())
def kernel(x, edge_index, edge_weight, W_xz, b_xz, W_hz, b_hz, W_xr, b_xr,
           W_hr, b_hr, W_xh, b_xh, W_hh, b_hh, W_lin, b_lin):
    n, f_in = x.shape
    out_len = W_lin.shape[1]
    bz = (b_xz + b_hz).reshape(1, -1)
    bh = (b_xh + b_hh).reshape(1, -1)
    bl = b_lin.reshape(1, -1)

    grid = (n // _BLK,)
    out = pl.pallas_call(
        _fused_gru_kernel,
        grid=grid,
        in_specs=[
            pl.BlockSpec((_BLK, f_in), lambda i: (i, 0)),
            pl.BlockSpec((f_in, W_xz.shape[1]), lambda i: (0, 0)),
            pl.BlockSpec((1, W_xz.shape[1]), lambda i: (0, 0)),
            pl.BlockSpec((f_in, W_xh.shape[1]), lambda i: (0, 0)),
            pl.BlockSpec((1, W_xh.shape[1]), lambda i: (0, 0)),
            pl.BlockSpec((W_lin.shape[0], out_len), lambda i: (0, 0)),
            pl.BlockSpec((1, out_len), lambda i: (0, 0)),
        ],
        out_specs=pl.BlockSpec((_BLK, out_len), lambda i: (i, 0)),
        out_shape=jax.ShapeDtypeStruct((n, out_len), x.dtype),
        compiler_params=pltpu.CompilerParams(
            dimension_semantics=("parallel",)),
    )(x, W_xz, bz, W_xh, bh, W_lin, bl)
    return (out,)
